# baseline (speedup 1.0000x reference)
import functools

import jax
import jax.numpy as jnp
from jax import lax
from jax.experimental import pallas as pl
from jax.experimental.pallas import tpu as pltpu
from jax.experimental.pallas import tpu_sc as plsc

_N = 10000
_IN = 128
_H = 64
_HEADS = 4
_OC = 16
_L = 8
_NG = 32
_E = 640000

_NPAD = 10240
_ET = _E + _N
_NW = 32
_EPW = 20480
_BB = 128
_NBLK = _EPW // _BB
_EPT = _NW * _EPW


def _make_sc_edge(npad, bb, nblk, epw, interpret=False):
    rows_per_tile = npad // 16
    assert rows_per_tile % bb == 0
    assert nblk % 2 == 0
    nz = rows_per_tile // bb
    ngrp = bb // 16

    mesh = plsc.VectorSubcoreMesh(
        core_axis_name="c", subcore_axis_name="s", num_cores=2,
        num_subcores=16)

    @functools.partial(
        pl.kernel,
        out_type=[
            jax.ShapeDtypeStruct((2, npad, 64), jnp.float32),
            jax.ShapeDtypeStruct((2, npad, 16), jnp.float32),
        ],
        mesh=mesh,
        scratch_types=[
            pltpu.VMEM((bb,), jnp.int32),
            pltpu.VMEM((bb,), jnp.int32),
            pltpu.VMEM((nblk, bb), jnp.int32),
            pltpu.VMEM((bb, 64), jnp.float32),
            pltpu.VMEM((bb, 64), jnp.float32),
            pltpu.VMEM((bb, 64), jnp.float32),
            pltpu.VMEM((bb, 16), jnp.float32),
            pltpu.VMEM((bb, 64), jnp.float32),
            pltpu.VMEM((bb, 64), jnp.float32),
            pltpu.VMEM((bb, 64), jnp.float32),
            pltpu.VMEM((bb, 16), jnp.float32),
            pltpu.VMEM((64,), jnp.float32),
            pltpu.VMEM_SHARED((npad, 64), jnp.float32),
            pltpu.VMEM_SHARED((npad, 16), jnp.float32),
        ] + [pltpu.SemaphoreType.DMA] * 10,
        compiler_params=pltpu.CompilerParams(
            needs_layout_passes=False, use_tc_tiling_on_sc=False),
        interpret=interpret,
    )
    def sc_edge(xl_h, xr_h, src_h, dst_h, att_h, num_h, den_h,
                src0, src1, dst_v, xl0, xr0, msg0, ex0, xl1, xr1, msg1,
                ex1, att_v, num_sp, den_sp,
                sxl0, sxr0, sm0, se0, si0, sxl1, sxr1, sm1, se1, si1):
        c = lax.axis_index("c")
        s = lax.axis_index("s")
        wid = c * 16 + s

        slots = ((src0, xl0, xr0, msg0, ex0, sxl0, sxr0, sm0, se0, si0),
                 (src1, xl1, xr1, msg1, ex1, sxl1, sxr1, sm1, se1, si1))

        z16 = jnp.zeros((16,), jnp.float32)

        def _zr(r, carry):
            for j in range(4):
                msg0[r, pl.ds(16 * j, 16)] = z16
            ex0[r, pl.ds(0, 16)] = z16
            ex1[r, pl.ds(0, 16)] = z16
            return carry

        lax.fori_loop(0, bb, _zr, 0)

        for t in range(nz):
            r0 = s * rows_per_tile + t * bb
            pltpu.sync_copy(msg0, num_sp.at[pl.ds(r0, bb)])
            pltpu.sync_copy(ex0, den_sp.at[pl.ds(r0, bb)])

        pltpu.sync_copy(dst_h.at[pl.ds(wid * nblk, nblk)], dst_v)
        pltpu.sync_copy(att_h, att_v)
        plsc.subcore_barrier()

        att4 = [att_v[pl.ds(16 * h, 16)] for h in range(4)]
        att_s = [att4[k // 16][k % 16] for k in range(64)]

        def _idx(slot, b):
            return pltpu.make_async_copy(
                src_h.at[wid * nblk + b], slots[slot][0], slots[slot][9])

        def _gather(slot, b):
            srcv, xlv, xrv = slots[slot][0], slots[slot][1], slots[slot][2]
            sxl, sxr = slots[slot][5], slots[slot][6]
            return (
                pltpu.make_async_copy(xl_h.at[srcv], xlv, sxl),
                pltpu.make_async_copy(xr_h.at[dst_v.at[b]], xrv, sxr),
            )

        def _issue_gather(slot, b):
            for d in _gather(slot, b):
                d.start()

        def _wait_gather(slot, b):
            for d in _gather(slot, b):
                d.wait()

        def _scatter(slot, b):
            msgv, exv = slots[slot][3], slots[slot][4]
            sm, se = slots[slot][7], slots[slot][8]
            return (
                pltpu.make_async_copy(msgv, num_sp.at[dst_v.at[b]], sm),
                pltpu.make_async_copy(exv, den_sp.at[dst_v.at[b]], se),
            )

        def _compute(slot, b):
            xlv, xrv, msgv, exv = slots[slot][1:5]

            def _grp(g, carry2):
                eidx = g * 16 + lax.iota(jnp.int32, 16)
                acc = [z16, z16, z16, z16]
                for k in range(64):
                    ck = jnp.full((16,), k, jnp.int32)
                    a = plsc.load_gather(xlv, [eidx, ck])
                    bv = plsc.load_gather(xrv, [eidx, ck])
                    u = a + bv
                    m = jnp.maximum(u, 0.2 * u)
                    acc[k // 16] = acc[k // 16] + m * att_s[k]
                exs = [jnp.exp(acc[h]) for h in range(4)]
                for h in range(4):
                    ch = jnp.full((16,), h, jnp.int32)
                    plsc.store_scatter(exv, [eidx, ch], exs[h])
                for k in range(64):
                    ck = jnp.full((16,), k, jnp.int32)
                    a = plsc.load_gather(xlv, [eidx, ck])
                    plsc.store_scatter(msgv, [eidx, ck], a * exs[k // 16])
                return carry2

            lax.fori_loop(0, ngrp, _grp, 0)

        pltpu.sync_copy(src_h.at[wid * nblk], src0)
        pltpu.sync_copy(src_h.at[wid * nblk + 1], src1)
        _issue_gather(0, 0)
        _issue_gather(1, 1)

        def _half(slot, j, b):
            _wait_gather(slot, b)

            @pl.when(b + 2 < nblk)
            def _():
                _idx(slot, b + 2).start()

            _compute(slot, b)

            @pl.when(b + 2 < nblk)
            def _():
                _idx(slot, b + 2).wait()
                _issue_gather(slot, b + 2)

            if slot == 0:
                @pl.when(j > 0)
                def _():
                    for d in _scatter(1, b - 1):
                        d.wait()
            else:
                for d in _scatter(0, b - 1):
                    d.wait()

            for d in _scatter(slot, b):
                d.start(add=True)

        def _blk(j, carry):
            _half(0, j, 2 * j)
            _half(1, j, 2 * j + 1)
            return carry

        lax.fori_loop(0, nblk // 2, _blk, 0)
        for d in _scatter(1, nblk - 1):
            d.wait()
        plsc.subcore_barrier()

        for t in range(nz):
            r0 = s * rows_per_tile + t * bb
            pltpu.sync_copy(num_sp.at[pl.ds(r0, bb)], num_h.at[c, pl.ds(r0, bb)])
            pltpu.sync_copy(den_sp.at[pl.ds(r0, bb)], den_h.at[c, pl.ds(r0, bb)])

    return sc_edge


def _dot_bf16(a, b):
    return jnp.dot(a.astype(jnp.bfloat16), b.astype(jnp.bfloat16),
                   preferred_element_type=jnp.float32)


def _pre_body(xs_ref, w_ref, b_ref, wl_ref, wr_ref, x0_ref, xl_ref, xr_ref):
    x0 = _dot_bf16(xs_ref[...], w_ref[...]) + b_ref[...]
    x0_ref[...] = x0
    xl_ref[...] = _dot_bf16(x0, wl_ref[...])
    xr_ref[...] = _dot_bf16(x0, wr_ref[...])


def _layer_body(num_ref, den_ref, xp_ref, cb_ref, g_ref, b_ref, wl_ref,
                wr_ref, xn_ref, xl_ref, xr_ref):
    num = num_ref[0] + num_ref[1]
    den = den_ref[0] + den_ref[1]
    h = jnp.concatenate(
        [num[:, 16 * hd:16 * (hd + 1)] / (den[:, hd:hd + 1] + 1e-16)
         for hd in range(4)], axis=1) + cb_ref[...]
    h1 = h[:_N]
    mu = jnp.mean(h1, 0, keepdims=True)
    var = jnp.mean((h1 - mu) ** 2, 0, keepdims=True)
    hn = (h - mu) / jnp.sqrt(var + 1e-5) * g_ref[...] + b_ref[...]
    hm = jnp.minimum(hn, 0.0)
    u = jnp.exp(hm)
    em1 = jnp.where(u == 1.0, hm,
                    (u - 1.0) * hm / jnp.log(jnp.maximum(u, 1e-30)))
    e = jnp.where(hn > 0, hn, em1)
    xn = xp_ref[...] + e
    xn_ref[...] = xn
    xl_ref[...] = _dot_bf16(xn, wl_ref[...])
    xr_ref[...] = _dot_bf16(xn, wr_ref[...])


def _final_body(x1_ref, x2_ref, b1_ref, b2_ref, f1w_ref, f1b_ref, f2w_ref,
                f2b_ref, f3w_ref, f3b_ref, out_ref):
    gi = lax.broadcasted_iota(jnp.int32, (_NG, _NPAD), 0)
    outs = []
    for bref, xref in ((b1_ref, x1_ref), (b2_ref, x2_ref)):
        m = (bref[...] == gi).astype(jnp.float32)
        x = xref[...]
        xhi = x.astype(jnp.bfloat16).astype(jnp.float32)
        xlo = x - xhi
        sacc = (jnp.dot(m, xhi, preferred_element_type=jnp.float32)
                + jnp.dot(m, xlo, preferred_element_type=jnp.float32))
        cnt = jnp.sum(m, axis=1, keepdims=True)
        outs.append(sacc / jnp.maximum(cnt, 1.0))
    cat = jnp.concatenate(outs, axis=1)
    hh = jnp.maximum(_dot_bf16(cat, f1w_ref[...]) + f1b_ref[...], 0.0)
    hh = jnp.maximum(_dot_bf16(hh, f2w_ref[...]) + f2b_ref[...], 0.0)
    out_ref[...] = _dot_bf16(hh, f3w_ref[...]) + f3b_ref[...]


def _sds(shape):
    return jax.ShapeDtypeStruct(shape, jnp.float32)


_tc_params = pltpu.CompilerParams(vmem_limit_bytes=64 * 1024 * 1024)

_tc_pre = pl.pallas_call(
    _pre_body,
    out_shape=[_sds((_NPAD, _H)), _sds((_NPAD, _H)), _sds((_NPAD, _H))],
    compiler_params=_tc_params)

_tc_layer = pl.pallas_call(
    _layer_body,
    out_shape=[_sds((_NPAD, _H)), _sds((_NPAD, _H)), _sds((_NPAD, _H))],
    compiler_params=_tc_params)

_tc_final = pl.pallas_call(_final_body, out_shape=[_sds((_NG, 1))],
                           compiler_params=_tc_params)


@functools.cache
def _sc_edge_cached():
    return _make_sc_edge(_NPAD, _BB, _NBLK, _EPW)


def kernel(x1, edge_index1, batch1, x2, edge_index2, batch2, node_W, node_b,
           Wl, Wr, att, conv_b, bn_g, bn_b, f1W, f1b, f2W, f2b, f3W, f3b):
    i32 = jnp.int32
    loop = jnp.arange(_N, dtype=i32)
    pad_e = _EPT - _ET

    def mk_edges(ei):
        src = jnp.concatenate([ei[0], loop, jnp.zeros((pad_e,), i32)])
        dst = jnp.concatenate([ei[1], loop, jnp.full((pad_e,), _N, i32)])
        return (src.reshape(_NW * _NBLK, _BB), dst.reshape(_NW * _NBLK, _BB))

    sc_edge = _sc_edge_cached()
    atts = [att[i].reshape(64) for i in range(_L)]

    outs = []
    for x_in, ei in ((x1, edge_index1), (x2, edge_index2)):
        src, dst = mk_edges(ei)
        xp = jnp.pad(x_in, ((0, _NPAD - _N), (0, 0)))
        x, xl, xr = _tc_pre(xp, node_W, node_b[None], Wl[0], Wr[0])
        for i in range(_L):
            num, den = sc_edge(xl, xr, src, dst, atts[i])
            nxt = (i + 1) % _L
            x, xl, xr = _tc_layer(num, den, x, conv_b[i][None],
                                  bn_g[i][None], bn_b[i][None], Wl[nxt],
                                  Wr[nxt])
        outs.append(x)

    b1r = jnp.pad(batch1, (0, _NPAD - _N), constant_values=_NG)[None, :]
    b2r = jnp.pad(batch2, (0, _NPAD - _N), constant_values=_NG)[None, :]
    (out,) = _tc_final(outs[0], outs[1], b1r, b2r, f1W, f1b[None], f2W,
                       f2b[None], f3W, f3b[None])
    return out


# speedup vs baseline: 1.1698x; 1.1698x over previous
import functools

import jax
import jax.numpy as jnp
from jax import lax
from jax.experimental import pallas as pl
from jax.experimental.pallas import tpu as pltpu
from jax.experimental.pallas import tpu_sc as plsc

_N = 10000
_IN = 128
_H = 64
_HEADS = 4
_OC = 16
_L = 8
_NG = 32
_E = 640000

_NPAD = 10240
_ET = _E + _N
_NW = 32
_EPW = 20480
_BB = 128
_NBLK = _EPW // _BB
_EPT = _NW * _EPW


def _make_sc_edge(npad, bb, nblk, epw, interpret=False):
    rows_per_tile = npad // 16
    assert rows_per_tile % bb == 0
    assert nblk % 2 == 0
    nz = rows_per_tile // bb
    ngrp = bb // 16

    mesh = plsc.VectorSubcoreMesh(
        core_axis_name="c", subcore_axis_name="s", num_cores=2,
        num_subcores=16)

    @functools.partial(
        pl.kernel,
        out_type=[
            jax.ShapeDtypeStruct((2, npad, 80), jnp.float32),
        ],
        mesh=mesh,
        scratch_types=[
            pltpu.VMEM((bb,), jnp.int32),
            pltpu.VMEM((bb,), jnp.int32),
            pltpu.VMEM((nblk, bb), jnp.int32),
            pltpu.VMEM((bb, 64), jnp.float32),
            pltpu.VMEM((bb, 64), jnp.float32),
            pltpu.VMEM((bb, 80), jnp.float32),
            pltpu.VMEM((bb, 64), jnp.float32),
            pltpu.VMEM((bb, 64), jnp.float32),
            pltpu.VMEM((bb, 80), jnp.float32),
            pltpu.VMEM((64,), jnp.float32),
            pltpu.VMEM_SHARED((npad, 80), jnp.float32),
        ] + [pltpu.SemaphoreType.DMA] * 8,
        compiler_params=pltpu.CompilerParams(
            needs_layout_passes=False, use_tc_tiling_on_sc=False),
        interpret=interpret,
    )
    def sc_edge(xl_h, xr_h, src_h, dst_h, att_h, num_h,
                src0, src1, dst_v, xl0, xr0, msg0, xl1, xr1, msg1,
                att_v, num_sp,
                sxl0, sxr0, sm0, si0, sxl1, sxr1, sm1, si1):
        c = lax.axis_index("c")
        s = lax.axis_index("s")
        wid = c * 16 + s

        slots = ((src0, xl0, xr0, msg0, sxl0, sxr0, sm0, si0),
                 (src1, xl1, xr1, msg1, sxl1, sxr1, sm1, si1))

        z16 = jnp.zeros((16,), jnp.float32)

        def _zr(r, carry):
            for j in range(5):
                msg0[r, pl.ds(16 * j, 16)] = z16
            msg1[r, pl.ds(64, 16)] = z16
            return carry

        lax.fori_loop(0, bb, _zr, 0)

        for t in range(nz):
            r0 = s * rows_per_tile + t * bb
            pltpu.sync_copy(msg0, num_sp.at[pl.ds(r0, bb)])

        pltpu.sync_copy(dst_h.at[pl.ds(wid * nblk, nblk)], dst_v)
        pltpu.sync_copy(att_h, att_v)
        plsc.subcore_barrier()

        att4 = [att_v[pl.ds(16 * h, 16)] for h in range(4)]
        att_s = [att4[k // 16][k % 16] for k in range(64)]

        def _idx(slot, b):
            return pltpu.make_async_copy(
                src_h.at[wid * nblk + b], slots[slot][0], slots[slot][7])

        def _gather(slot, b):
            srcv, xlv, xrv = slots[slot][0], slots[slot][1], slots[slot][2]
            sxl, sxr = slots[slot][4], slots[slot][5]
            return (
                pltpu.make_async_copy(xl_h.at[srcv], xlv, sxl),
                pltpu.make_async_copy(xr_h.at[dst_v.at[b]], xrv, sxr),
            )

        def _issue_gather(slot, b):
            for d in _gather(slot, b):
                d.start()

        def _wait_gather(slot, b):
            for d in _gather(slot, b):
                d.wait()

        def _scatter(slot, b):
            msgv, sm = slots[slot][3], slots[slot][6]
            return (
                pltpu.make_async_copy(msgv, num_sp.at[dst_v.at[b]], sm),
            )

        def _compute(slot, b):
            xlv, xrv, msgv = slots[slot][1:4]

            def _grp(g, carry2):
                eidx = g * 16 + lax.iota(jnp.int32, 16)
                acc = [z16, z16, z16, z16]
                for k in range(64):
                    ck = jnp.full((16,), k, jnp.int32)
                    a = plsc.load_gather(xlv, [eidx, ck])
                    bv = plsc.load_gather(xrv, [eidx, ck])
                    u = a + bv
                    m = jnp.maximum(u, 0.2 * u)
                    acc[k // 16] = acc[k // 16] + m * att_s[k]
                exs = [jnp.exp(acc[h]) for h in range(4)]
                for h in range(4):
                    ch = jnp.full((16,), 64 + h, jnp.int32)
                    plsc.store_scatter(msgv, [eidx, ch], exs[h])
                for k in range(64):
                    ck = jnp.full((16,), k, jnp.int32)
                    a = plsc.load_gather(xlv, [eidx, ck])
                    plsc.store_scatter(msgv, [eidx, ck], a * exs[k // 16])
                return carry2

            lax.fori_loop(0, ngrp, _grp, 0)

        pltpu.sync_copy(src_h.at[wid * nblk], src0)
        pltpu.sync_copy(src_h.at[wid * nblk + 1], src1)
        _issue_gather(0, 0)
        _issue_gather(1, 1)

        def _half(slot, j, b):
            _wait_gather(slot, b)

            @pl.when(b + 2 < nblk)
            def _():
                _idx(slot, b + 2).start()

            _compute(slot, b)

            @pl.when(b + 2 < nblk)
            def _():
                _idx(slot, b + 2).wait()
                _issue_gather(slot, b + 2)

            if slot == 0:
                @pl.when(j > 0)
                def _():
                    for d in _scatter(1, b - 1):
                        d.wait()
            else:
                for d in _scatter(0, b - 1):
                    d.wait()

            for d in _scatter(slot, b):
                d.start(add=True)

        def _blk(j, carry):
            _half(0, j, 2 * j)
            _half(1, j, 2 * j + 1)
            return carry

        lax.fori_loop(0, nblk // 2, _blk, 0)
        for d in _scatter(1, nblk - 1):
            d.wait()
        plsc.subcore_barrier()

        for t in range(nz):
            r0 = s * rows_per_tile + t * bb
            pltpu.sync_copy(num_sp.at[pl.ds(r0, bb)], num_h.at[c, pl.ds(r0, bb)])

    return sc_edge


def _dot_bf16(a, b):
    return jnp.dot(a.astype(jnp.bfloat16), b.astype(jnp.bfloat16),
                   preferred_element_type=jnp.float32)


def _pre_body(xs_ref, w_ref, b_ref, wl_ref, wr_ref, x0_ref, xl_ref, xr_ref):
    x0 = _dot_bf16(xs_ref[...], w_ref[...]) + b_ref[...]
    x0_ref[...] = x0
    xl_ref[...] = _dot_bf16(x0, wl_ref[...])
    xr_ref[...] = _dot_bf16(x0, wr_ref[...])


def _layer_body(num_ref, xp_ref, cb_ref, g_ref, b_ref, wl_ref,
                wr_ref, xn_ref, xl_ref, xr_ref):
    num = num_ref[0] + num_ref[1]
    h = jnp.concatenate(
        [num[:, 16 * hd:16 * (hd + 1)] / (num[:, 64 + hd:65 + hd] + 1e-16)
         for hd in range(4)], axis=1) + cb_ref[...]
    h1 = h[:_N]
    mu = jnp.mean(h1, 0, keepdims=True)
    var = jnp.mean((h1 - mu) ** 2, 0, keepdims=True)
    hn = (h - mu) / jnp.sqrt(var + 1e-5) * g_ref[...] + b_ref[...]
    hm = jnp.minimum(hn, 0.0)
    u = jnp.exp(hm)
    em1 = jnp.where(u == 1.0, hm,
                    (u - 1.0) * hm / jnp.log(jnp.maximum(u, 1e-30)))
    e = jnp.where(hn > 0, hn, em1)
    xn = xp_ref[...] + e
    xn_ref[...] = xn
    xl_ref[...] = _dot_bf16(xn, wl_ref[...])
    xr_ref[...] = _dot_bf16(xn, wr_ref[...])


def _final_body(x1_ref, x2_ref, b1_ref, b2_ref, f1w_ref, f1b_ref, f2w_ref,
                f2b_ref, f3w_ref, f3b_ref, out_ref):
    gi = lax.broadcasted_iota(jnp.int32, (_NG, _NPAD), 0)
    outs = []
    for bref, xref in ((b1_ref, x1_ref), (b2_ref, x2_ref)):
        m = (bref[...] == gi).astype(jnp.float32)
        x = xref[...]
        xhi = x.astype(jnp.bfloat16).astype(jnp.float32)
        xlo = x - xhi
        sacc = (jnp.dot(m, xhi, preferred_element_type=jnp.float32)
                + jnp.dot(m, xlo, preferred_element_type=jnp.float32))
        cnt = jnp.sum(m, axis=1, keepdims=True)
        outs.append(sacc / jnp.maximum(cnt, 1.0))
    cat = jnp.concatenate(outs, axis=1)
    hh = jnp.maximum(_dot_bf16(cat, f1w_ref[...]) + f1b_ref[...], 0.0)
    hh = jnp.maximum(_dot_bf16(hh, f2w_ref[...]) + f2b_ref[...], 0.0)
    out_ref[...] = _dot_bf16(hh, f3w_ref[...]) + f3b_ref[...]


def _sds(shape):
    return jax.ShapeDtypeStruct(shape, jnp.float32)


_tc_params = pltpu.CompilerParams(vmem_limit_bytes=64 * 1024 * 1024)

_tc_pre = pl.pallas_call(
    _pre_body,
    out_shape=[_sds((_NPAD, _H)), _sds((_NPAD, _H)), _sds((_NPAD, _H))],
    compiler_params=_tc_params)

_tc_layer = pl.pallas_call(
    _layer_body,
    out_shape=[_sds((_NPAD, _H)), _sds((_NPAD, _H)), _sds((_NPAD, _H))],
    compiler_params=_tc_params)

_tc_final = pl.pallas_call(_final_body, out_shape=[_sds((_NG, 1))],
                           compiler_params=_tc_params)


@functools.cache
def _sc_edge_cached():
    return _make_sc_edge(_NPAD, _BB, _NBLK, _EPW)


def kernel(x1, edge_index1, batch1, x2, edge_index2, batch2, node_W, node_b,
           Wl, Wr, att, conv_b, bn_g, bn_b, f1W, f1b, f2W, f2b, f3W, f3b):
    i32 = jnp.int32
    loop = jnp.arange(_N, dtype=i32)
    pad_e = _EPT - _ET

    def mk_edges(ei):
        src = jnp.concatenate([ei[0], loop, jnp.zeros((pad_e,), i32)])
        dst = jnp.concatenate([ei[1], loop, jnp.full((pad_e,), _N, i32)])
        return (src.reshape(_NW * _NBLK, _BB), dst.reshape(_NW * _NBLK, _BB))

    sc_edge = _sc_edge_cached()
    atts = [att[i].reshape(64) for i in range(_L)]

    outs = []
    for x_in, ei in ((x1, edge_index1), (x2, edge_index2)):
        src, dst = mk_edges(ei)
        xp = jnp.pad(x_in, ((0, _NPAD - _N), (0, 0)))
        x, xl, xr = _tc_pre(xp, node_W, node_b[None], Wl[0], Wr[0])
        for i in range(_L):
            (num,) = sc_edge(xl, xr, src, dst, atts[i])
            nxt = (i + 1) % _L
            x, xl, xr = _tc_layer(num, x, conv_b[i][None],
                                  bn_g[i][None], bn_b[i][None], Wl[nxt],
                                  Wr[nxt])
        outs.append(x)

    b1r = jnp.pad(batch1, (0, _NPAD - _N), constant_values=_NG)[None, :]
    b2r = jnp.pad(batch2, (0, _NPAD - _N), constant_values=_NG)[None, :]
    (out,) = _tc_final(outs[0], outs[1], b1r, b2r, f1W, f1b[None], f2W,
                       f2b[None], f3W, f3b[None])
    return out
